# baseline (speedup 1.0000x reference)
import jax
import jax.numpy as jnp
from jax import lax
from jax.experimental import pallas as pl
from jax.experimental.pallas import tpu as pltpu
from jax.experimental.pallas import tpu_sc as plsc

L = 16
N = 576
NCHUNK = N // L
K = 24
NEG = -3.0e38
BIG = 4096


def _iota():
    return lax.iota(jnp.int32, L)


def _splat_sum_i(v, ref):
    io = _iota()
    for sh in (1, 2, 4, 8):
        ref[pl.ds(0, L)] = v
        g = plsc.load_gather(ref, [jnp.bitwise_xor(io, sh)])
        v = v + g
    return v


def _splat_min_i(v, ref):
    io = _iota()
    for sh in (1, 2, 4, 8):
        ref[pl.ds(0, L)] = v
        g = plsc.load_gather(ref, [jnp.bitwise_xor(io, sh)])
        v = jnp.minimum(v, g)
    return v


def _sortd(v, descending=True):
    k, _ = plsc.sort_key_val(v, v, descending=descending)
    return k


def _merge16(s1, s2):
    r = jnp.flip(s2)
    hi = jnp.maximum(s1, r)
    lo = jnp.minimum(s1, r)
    return _sortd(hi), _sortd(lo)


def _merge32(x, y):
    x1, x2 = x
    y1, y2 = y
    t1 = jnp.maximum(x1, jnp.flip(y2))
    t2 = jnp.maximum(x2, jnp.flip(y1))
    return _merge16(_sortd(t1), _sortd(t2))


def _top32_tree(chunks):
    sorted_chunks = [_sortd(c) for c in chunks]
    nodes = [_merge16(sorted_chunks[2 * i], sorted_chunks[2 * i + 1])
             for i in range(len(sorted_chunks) // 2)]
    if len(sorted_chunks) % 2:
        s = sorted_chunks[-1]
        pad = jnp.full((L,), s.dtype.type(0), s.dtype)
        nodes.append((s, pad))
    while len(nodes) > 1:
        nxt = [_merge32(nodes[2 * i], nodes[2 * i + 1])
               for i in range(len(nodes) // 2)]
        if len(nodes) % 2:
            nxt.append(nodes[-1])
        nodes = nxt
    return nodes[0]


ROWSTRIDE = 577 * 577


def _sc_body(x_hbm, out_hbm, raw_v, rows_v, cnt_v, tot_v, h_v, e_v,
             out_v, four_v, redi_v, shared):
    c = lax.axis_index("c")
    s = lax.axis_index("s")
    b = 4 * c + s // 4
    hg = s % 4
    r0 = b * 12 + 3 * hg

    io = _iota()
    pltpu.sync_copy(x_hbm.at[pl.ds(r0 * N, 3 * N)], rows_v)

    zero16 = jnp.zeros((L,), jnp.float32)
    for j in range(NCHUNK):
        cnt_v[pl.ds(j * L, L)] = zero16

    for rr in range(3):
        chunks = [rows_v[pl.ds(rr * N + j * L, L)] for j in range(NCHUNK)]
        r1, r2 = _top32_tree(chunks)
        thr = r2[K - L - 1]
        tsp = jnp.full((L,), thr, jnp.float32)

        gcnt = (jnp.where(r1 > tsp, 1, 0) + jnp.where(r2 > tsp, 1, 0))
        need = K - _splat_sum_i(gcnt, redi_v)[0]

        bidx0 = jnp.full((L,), BIG, jnp.int32)
        for j in range(NCHUNK):
            hit = rows_v[pl.ds(rr * N + j * L, L)] == tsp
            bidx0 = jnp.minimum(bidx0, jnp.where(hit, io + j * L, BIG))
        cut0 = _splat_min_i(bidx0, redi_v)[0]

        def fcond(st):
            i, _ = st
            return i < need - 1

        def fbody(st):
            i, cc = st
            csp = jnp.full((L,), cc, jnp.int32)
            bidx = jnp.full((L,), BIG, jnp.int32)
            for j in range(NCHUNK):
                v = rows_v[pl.ds(rr * N + j * L, L)]
                p = io + j * L
                hit = jnp.logical_and(v == tsp, p > csp)
                bidx = jnp.minimum(bidx, jnp.where(hit, p, BIG))
            return i + 1, _splat_min_i(bidx, redi_v)[0]

        _, cut = lax.while_loop(fcond, fbody, (jnp.int32(0), cut0))
        csp = jnp.full((L,), cut, jnp.int32)

        for j in range(NCHUNK):
            v = rows_v[pl.ds(rr * N + j * L, L)]
            p = io + j * L
            memb = jnp.logical_or(
                v > tsp, jnp.logical_and(v == tsp, p <= csp))
            cnt_v[pl.ds(j * L, L)] = (cnt_v[pl.ds(j * L, L)]
                                      + jnp.where(memb, 1.0, 0.0))

    pltpu.sync_copy(cnt_v, shared.at[pl.ds(s * N, N)])
    plsc.subcore_barrier()

    @pl.when(s % 4 == 0)
    def _leader():
        pltpu.sync_copy(shared.at[pl.ds(s * N, 4 * N)], four_v)
        for j in range(NCHUNK):
            base = j * L
            t0 = four_v[pl.ds(base, L)] + four_v[pl.ds(N + base, L)]
            t1 = (four_v[pl.ds(2 * N + base, L)]
                  + four_v[pl.ds(3 * N + base, L)])
            tot_v[pl.ds(base, L)] = t0 + t1

        for j in range(NCHUNK):
            base = j * L
            p = io + base
            xc = p % 24
            t = tot_v[pl.ds(base, L)]
            left = plsc.load_gather(tot_v, [jnp.maximum(p - 1, 0)])
            right = plsc.load_gather(tot_v, [jnp.minimum(p + 1, N - 1)])
            h_v[pl.ds(base, L)] = (2.0 * t
                                   + jnp.where(xc > 0, left, 0.0)
                                   + jnp.where(xc < 23, right, 0.0))

        for j in range(NCHUNK):
            base = j * L
            p = io + base
            t = h_v[pl.ds(base, L)]
            up = plsc.load_gather(h_v, [jnp.maximum(p - 24, 0)])
            dn = plsc.load_gather(h_v, [jnp.minimum(p + 24, N - 1)])
            e_v[pl.ds(base, L)] = (2.0 * t
                                   + jnp.where(p >= 24, up, 0.0)
                                   + jnp.where(p < N - 24, dn, 0.0))

        keys = []
        for j in range(NCHUNK):
            base = j * L
            p = io + base
            ei = e_v[pl.ds(base, L)].astype(jnp.int32)
            keys.append(jnp.bitwise_or(jnp.left_shift(ei, 10), 1023 - p))
        k1, k2 = _top32_tree(keys)
        o0 = 1024 - jnp.bitwise_and(k1, 1023)
        o1 = 1024 - jnp.bitwise_and(k2, 1023)
        out_v[pl.ds(0, L)] = o0
        out_v[pl.ds(L, L)] = o1
        pltpu.sync_copy(out_v, out_hbm.at[pl.ds(b * 32, 32)])


@jax.jit
def kernel(x):
    bb, hh, mm, _ = x.shape
    mesh = plsc.VectorSubcoreMesh(core_axis_name="c", subcore_axis_name="s")
    run = pl.kernel(
        _sc_body,
        mesh=mesh,
        compiler_params=pltpu.CompilerParams(needs_layout_passes=False),
        out_type=jax.ShapeDtypeStruct((bb * 32,), jnp.int32),
        scratch_types=[
            pltpu.VMEM((8,), jnp.float32),
            pltpu.VMEM((3 * N,), jnp.float32),
            pltpu.VMEM((N,), jnp.float32),
            pltpu.VMEM((N,), jnp.float32),
            pltpu.VMEM((N,), jnp.float32),
            pltpu.VMEM((N,), jnp.float32),
            pltpu.VMEM((32,), jnp.int32),
            pltpu.VMEM((4 * N,), jnp.float32),
            pltpu.VMEM((L,), jnp.int32),
            pltpu.VMEM_SHARED((16 * N,), jnp.float32),
        ],
    )
    score = x[:, :, 0, 1:].reshape(bb * hh * (mm - 1))
    return run(score).reshape(bb, 32)[:, :K]


# speedup vs baseline: 1.0406x; 1.0406x over previous
import jax
import jax.numpy as jnp
from jax import lax
from jax.experimental import pallas as pl
from jax.experimental.pallas import tpu as pltpu
from jax.experimental.pallas import tpu_sc as plsc

L = 16
N = 576
NCHUNK = N // L
K = 24
NEG = -3.0e38
BIG = 4096


def _iota():
    return lax.iota(jnp.int32, L)


def _splat_sum_i(v, ref):
    io = _iota()
    for sh in (1, 2, 4, 8):
        ref[pl.ds(0, L)] = v
        g = plsc.load_gather(ref, [jnp.bitwise_xor(io, sh)])
        v = v + g
    return v


def _splat_min_i(v, ref):
    io = _iota()
    for sh in (1, 2, 4, 8):
        ref[pl.ds(0, L)] = v
        g = plsc.load_gather(ref, [jnp.bitwise_xor(io, sh)])
        v = jnp.minimum(v, g)
    return v


def _sortd(v, descending=True):
    k, _ = plsc.sort_key_val(v, v, descending=descending)
    return k


def _merge16(s1, s2):
    r = jnp.flip(s2)
    hi = jnp.maximum(s1, r)
    lo = jnp.minimum(s1, r)
    return _sortd(hi), _sortd(lo)


def _merge32(x, y):
    x1, x2 = x
    y1, y2 = y
    t1 = jnp.maximum(x1, jnp.flip(y2))
    t2 = jnp.maximum(x2, jnp.flip(y1))
    return _merge16(_sortd(t1), _sortd(t2))


def _top32_tree(chunks):
    sorted_chunks = [_sortd(c) for c in chunks]
    nodes = [_merge16(sorted_chunks[2 * i], sorted_chunks[2 * i + 1])
             for i in range(len(sorted_chunks) // 2)]
    if len(sorted_chunks) % 2:
        s = sorted_chunks[-1]
        pad = jnp.full((L,), s.dtype.type(0), s.dtype)
        nodes.append((s, pad))
    while len(nodes) > 1:
        nxt = [_merge32(nodes[2 * i], nodes[2 * i + 1])
               for i in range(len(nodes) // 2)]
        if len(nodes) % 2:
            nxt.append(nodes[-1])
        nodes = nxt
    return nodes[0]


ROWSTRIDE = 577 * 577


def _sc_body(x_hbm, out_hbm, raw_v, rows_v, cnt_v, tot_v, h_v, e_v,
             out_v, four_v, redi_v, shared):
    c = lax.axis_index("c")
    s = lax.axis_index("s")
    b = 4 * c + s // 4
    hg = s % 4
    r0 = b * 12 + 3 * hg

    io = _iota()
    pltpu.sync_copy(x_hbm.at[pl.ds(r0 * N, 3 * N)], rows_v)

    zero16 = jnp.zeros((L,), jnp.float32)
    for j in range(NCHUNK):
        cnt_v[pl.ds(j * L, L)] = zero16

    for rr in range(3):
        chunks = [rows_v[pl.ds(rr * N + j * L, L)] for j in range(NCHUNK)]
        r1, r2 = _top32_tree(chunks)
        thr = r2[K - L - 1]
        tsp = jnp.full((L,), thr, jnp.float32)

        gcnt = (jnp.where(r1 > tsp, 1, 0) + jnp.where(r2 > tsp, 1, 0))
        need = K - _splat_sum_i(gcnt, redi_v)[0]

        def fcond(st):
            i, _ = st
            return i < need

        def fbody(st):
            i, cc = st
            csp = jnp.full((L,), cc, jnp.int32)
            bidx = jnp.full((L,), BIG, jnp.int32)
            for j in range(NCHUNK):
                v = rows_v[pl.ds(rr * N + j * L, L)]
                p = io + j * L
                hit = jnp.logical_and(v == tsp, p > csp)
                bidx = jnp.minimum(bidx, jnp.where(hit, p, BIG))
            return i + 1, _splat_min_i(bidx, redi_v)[0]

        _, cut = lax.while_loop(fcond, fbody, (jnp.int32(0), jnp.int32(-1)))
        csp = jnp.full((L,), cut, jnp.int32)

        for j in range(NCHUNK):
            v = rows_v[pl.ds(rr * N + j * L, L)]
            p = io + j * L
            memb = jnp.logical_or(
                v > tsp, jnp.logical_and(v == tsp, p <= csp))
            cnt_v[pl.ds(j * L, L)] = (cnt_v[pl.ds(j * L, L)]
                                      + jnp.where(memb, 1.0, 0.0))

    pltpu.sync_copy(cnt_v, shared.at[pl.ds(s * N, N)])
    plsc.subcore_barrier()

    @pl.when(s % 4 == 0)
    def _leader():
        pltpu.sync_copy(shared.at[pl.ds(s * N, 4 * N)], four_v)
        for j in range(NCHUNK):
            base = j * L
            t0 = four_v[pl.ds(base, L)] + four_v[pl.ds(N + base, L)]
            t1 = (four_v[pl.ds(2 * N + base, L)]
                  + four_v[pl.ds(3 * N + base, L)])
            tot_v[pl.ds(base, L)] = t0 + t1

        for j in range(NCHUNK):
            base = j * L
            p = io + base
            xc = p % 24
            t = tot_v[pl.ds(base, L)]
            left = plsc.load_gather(tot_v, [jnp.maximum(p - 1, 0)])
            right = plsc.load_gather(tot_v, [jnp.minimum(p + 1, N - 1)])
            h_v[pl.ds(base, L)] = (2.0 * t
                                   + jnp.where(xc > 0, left, 0.0)
                                   + jnp.where(xc < 23, right, 0.0))

        for j in range(NCHUNK):
            base = j * L
            p = io + base
            t = h_v[pl.ds(base, L)]
            up = plsc.load_gather(h_v, [jnp.maximum(p - 24, 0)])
            dn = plsc.load_gather(h_v, [jnp.minimum(p + 24, N - 1)])
            e_v[pl.ds(base, L)] = (2.0 * t
                                   + jnp.where(p >= 24, up, 0.0)
                                   + jnp.where(p < N - 24, dn, 0.0))

        keys = []
        for j in range(NCHUNK):
            base = j * L
            p = io + base
            ei = e_v[pl.ds(base, L)].astype(jnp.int32)
            keys.append(jnp.bitwise_or(jnp.left_shift(ei, 10), 1023 - p))
        k1, k2 = _top32_tree(keys)
        o0 = 1024 - jnp.bitwise_and(k1, 1023)
        o1 = 1024 - jnp.bitwise_and(k2, 1023)
        out_v[pl.ds(0, L)] = o0
        out_v[pl.ds(L, L)] = o1
        pltpu.sync_copy(out_v, out_hbm.at[pl.ds(b * 32, 32)])


@jax.jit
def kernel(x):
    bb, hh, mm, _ = x.shape
    mesh = plsc.VectorSubcoreMesh(core_axis_name="c", subcore_axis_name="s")
    run = pl.kernel(
        _sc_body,
        mesh=mesh,
        compiler_params=pltpu.CompilerParams(needs_layout_passes=False),
        out_type=jax.ShapeDtypeStruct((bb * 32,), jnp.int32),
        scratch_types=[
            pltpu.VMEM((8,), jnp.float32),
            pltpu.VMEM((3 * N,), jnp.float32),
            pltpu.VMEM((N,), jnp.float32),
            pltpu.VMEM((N,), jnp.float32),
            pltpu.VMEM((N,), jnp.float32),
            pltpu.VMEM((N,), jnp.float32),
            pltpu.VMEM((32,), jnp.int32),
            pltpu.VMEM((4 * N,), jnp.float32),
            pltpu.VMEM((L,), jnp.int32),
            pltpu.VMEM_SHARED((16 * N,), jnp.float32),
        ],
    )
    score = x[:, :, 0, 1:].reshape(bb * hh * (mm - 1))
    return run(score).reshape(bb, 32)[:, :K]


# speedup vs baseline: 1.0408x; 1.0002x over previous
import jax
import jax.numpy as jnp
from jax import lax
from jax.experimental import pallas as pl
from jax.experimental.pallas import tpu as pltpu
from jax.experimental.pallas import tpu_sc as plsc

L = 16
N = 576
NCHUNK = N // L
K = 24
NEG = -3.0e38
BIG = 4096


def _iota():
    return lax.iota(jnp.int32, L)


def _splat_sum_i(v, ref):
    io = _iota()
    for sh in (1, 2, 4, 8):
        ref[pl.ds(0, L)] = v
        g = plsc.load_gather(ref, [jnp.bitwise_xor(io, sh)])
        v = v + g
    return v


def _splat_min_i(v, ref):
    io = _iota()
    for sh in (1, 2, 4, 8):
        ref[pl.ds(0, L)] = v
        g = plsc.load_gather(ref, [jnp.bitwise_xor(io, sh)])
        v = jnp.minimum(v, g)
    return v


def _sortd(v, descending=True):
    k, _ = plsc.sort_key_val(v, v, descending=descending)
    return k


def _merge16(s1, s2):
    r = jnp.flip(s2)
    hi = jnp.maximum(s1, r)
    lo = jnp.minimum(s1, r)
    return _sortd(hi), _sortd(lo)


def _merge32(x, y):
    x1, x2 = x
    y1, y2 = y
    t1 = jnp.maximum(x1, jnp.flip(y2))
    t2 = jnp.maximum(x2, jnp.flip(y1))
    return _merge16(_sortd(t1), _sortd(t2))


def _top32_tree(chunks):
    sorted_chunks = [_sortd(c) for c in chunks]
    nodes = [_merge16(sorted_chunks[2 * i], sorted_chunks[2 * i + 1])
             for i in range(len(sorted_chunks) // 2)]
    if len(sorted_chunks) % 2:
        s = sorted_chunks[-1]
        pad = jnp.full((L,), s.dtype.type(0), s.dtype)
        nodes.append((s, pad))
    while len(nodes) > 1:
        nxt = [_merge32(nodes[2 * i], nodes[2 * i + 1])
               for i in range(len(nodes) // 2)]
        if len(nodes) % 2:
            nxt.append(nodes[-1])
        nodes = nxt
    return nodes[0]


ROWSTRIDE = 577 * 577


def _sc_body(x_hbm, out_hbm, raw_v, rows_v, cnt_v, tot_v, h_v, e_v,
             out_v, four_v, redi_v, shared):
    c = lax.axis_index("c")
    s = lax.axis_index("s")
    b = 4 * c + s // 4
    hg = s % 4
    r0 = b * 12 + 3 * hg

    io = _iota()
    pltpu.sync_copy(x_hbm.at[pl.ds(r0 * N, 3 * N)], rows_v)

    zero16 = jnp.zeros((L,), jnp.float32)
    for j in range(NCHUNK):
        cnt_v[pl.ds(j * L, L)] = zero16

    for rr in range(3):
        chunks = [rows_v[pl.ds(rr * N + j * L, L)] for j in range(NCHUNK)]
        r1, r2 = _top32_tree(chunks)
        thr = r2[K - L - 1]
        tsp = jnp.full((L,), thr, jnp.float32)

        gcnt = (jnp.where(r1 > tsp, 1, 0) + jnp.where(r2 > tsp, 1, 0))
        need = K - _splat_sum_i(gcnt, redi_v)[0]

        def fcond(st):
            i, _ = st
            return i < need

        def fbody(st):
            i, cc = st
            csp = jnp.full((L,), cc, jnp.int32)
            bidx = jnp.full((L,), BIG, jnp.int32)
            for j in range(NCHUNK):
                v = rows_v[pl.ds(rr * N + j * L, L)]
                p = io + j * L
                hit = jnp.logical_and(v == tsp, p > csp)
                bidx = jnp.minimum(bidx, jnp.where(hit, p, BIG))
            return i + 1, _splat_min_i(bidx, redi_v)[0]

        _, cut = lax.while_loop(fcond, fbody, (jnp.int32(0), jnp.int32(-1)))
        csp = jnp.full((L,), cut, jnp.int32)

        for j in range(NCHUNK):
            v = rows_v[pl.ds(rr * N + j * L, L)]
            p = io + j * L
            memb = jnp.logical_or(
                v > tsp, jnp.logical_and(v == tsp, p <= csp))
            cnt_v[pl.ds(j * L, L)] = (cnt_v[pl.ds(j * L, L)]
                                      + jnp.where(memb, 1.0, 0.0))

    pltpu.sync_copy(cnt_v, shared.at[pl.ds(s * N, N)])
    plsc.subcore_barrier()

    @pl.when(s % 4 == 0)
    def _leader():
        pltpu.sync_copy(shared.at[pl.ds(s * N, 4 * N)], four_v)
        for j in range(NCHUNK):
            base = j * L
            t0 = four_v[pl.ds(base, L)] + four_v[pl.ds(N + base, L)]
            t1 = (four_v[pl.ds(2 * N + base, L)]
                  + four_v[pl.ds(3 * N + base, L)])
            tot_v[pl.ds(base, L)] = t0 + t1

        for j in range(NCHUNK):
            base = j * L
            p = io + base
            xc = p % 24
            t = tot_v[pl.ds(base, L)]
            left = plsc.load_gather(tot_v, [jnp.maximum(p - 1, 0)])
            right = plsc.load_gather(tot_v, [jnp.minimum(p + 1, N - 1)])
            h_v[pl.ds(base, L)] = (2.0 * t
                                   + jnp.where(xc > 0, left, 0.0)
                                   + jnp.where(xc < 23, right, 0.0))

        for j in range(NCHUNK):
            base = j * L
            p = io + base
            t = h_v[pl.ds(base, L)]
            up = plsc.load_gather(h_v, [jnp.maximum(p - 24, 0)])
            dn = plsc.load_gather(h_v, [jnp.minimum(p + 24, N - 1)])
            e_v[pl.ds(base, L)] = (2.0 * t
                                   + jnp.where(p >= 24, up, 0.0)
                                   + jnp.where(p < N - 24, dn, 0.0))

        keys = []
        for j in range(NCHUNK):
            base = j * L
            p = io + base
            ei = e_v[pl.ds(base, L)].astype(jnp.int32)
            keys.append(jnp.bitwise_or(jnp.left_shift(ei, 10), 1023 - p))
        k1, k2 = _top32_tree(keys)
        o0 = 1024 - jnp.bitwise_and(k1, 1023)
        o1 = 1024 - jnp.bitwise_and(k2, 1023)
        out_v[pl.ds(0, L)] = o0
        out_v[pl.ds(L, L)] = o1
        pltpu.sync_copy(out_v, out_hbm.at[pl.ds(b * 32, 32)])


@jax.jit
def kernel(x):
    bb, hh, mm, _ = x.shape
    mesh = plsc.VectorSubcoreMesh(core_axis_name="c", subcore_axis_name="s")
    run = pl.kernel(
        _sc_body,
        mesh=mesh,
        compiler_params=pltpu.CompilerParams(needs_layout_passes=False, allow_input_fusion=[True]),
        out_type=jax.ShapeDtypeStruct((bb * 32,), jnp.int32),
        scratch_types=[
            pltpu.VMEM((8,), jnp.float32),
            pltpu.VMEM((3 * N,), jnp.float32),
            pltpu.VMEM((N,), jnp.float32),
            pltpu.VMEM((N,), jnp.float32),
            pltpu.VMEM((N,), jnp.float32),
            pltpu.VMEM((N,), jnp.float32),
            pltpu.VMEM((32,), jnp.int32),
            pltpu.VMEM((4 * N,), jnp.float32),
            pltpu.VMEM((L,), jnp.int32),
            pltpu.VMEM_SHARED((16 * N,), jnp.float32),
        ],
    )
    score = x[:, :, 0, 1:].reshape(bb * hh * (mm - 1))
    return run(score).reshape(bb, 32)[:, :K]


# speedup vs baseline: 1.0458x; 1.0048x over previous
import jax
import jax.numpy as jnp
from jax import lax
from jax.experimental import pallas as pl
from jax.experimental.pallas import tpu as pltpu
from jax.experimental.pallas import tpu_sc as plsc

L = 16
N = 576
NCHUNK = N // L
K = 24
NEG = -3.0e38
BIG = 4096


def _iota():
    return lax.iota(jnp.int32, L)


def _splat_sum_i(v, ref):
    io = _iota()
    for sh in (1, 2, 4, 8):
        ref[pl.ds(0, L)] = v
        g = plsc.load_gather(ref, [jnp.bitwise_xor(io, sh)])
        v = v + g
    return v


def _splat_min_i(v, ref):
    io = _iota()
    for sh in (1, 2, 4, 8):
        ref[pl.ds(0, L)] = v
        g = plsc.load_gather(ref, [jnp.bitwise_xor(io, sh)])
        v = jnp.minimum(v, g)
    return v


def _sortd(v, descending=True):
    k, _ = plsc.sort_key_val(v, v, descending=descending)
    return k


def _merge16(s1, s2):
    r = jnp.flip(s2)
    hi = jnp.maximum(s1, r)
    lo = jnp.minimum(s1, r)
    return _sortd(hi), _sortd(lo)


def _merge32(x, y):
    x1, x2 = x
    y1, y2 = y
    t1 = jnp.maximum(x1, jnp.flip(y2))
    t2 = jnp.maximum(x2, jnp.flip(y1))
    return _merge16(_sortd(t1), _sortd(t2))


def _top32_tree(chunks):
    sorted_chunks = [_sortd(c) for c in chunks]
    nodes = [_merge16(sorted_chunks[2 * i], sorted_chunks[2 * i + 1])
             for i in range(len(sorted_chunks) // 2)]
    if len(sorted_chunks) % 2:
        s = sorted_chunks[-1]
        pad = jnp.full((L,), s.dtype.type(0), s.dtype)
        nodes.append((s, pad))
    while len(nodes) > 1:
        nxt = [_merge32(nodes[2 * i], nodes[2 * i + 1])
               for i in range(len(nodes) // 2)]
        if len(nodes) % 2:
            nxt.append(nodes[-1])
        nodes = nxt
    return nodes[0]


ROWSTRIDE = 577 * 577


def _sc_body(x_hbm, out_hbm, raw_v, rows_v, cnt_v, tot_v, h_v, e_v,
             out_v, four_v, redi_v, shared):
    c = lax.axis_index("c")
    s = lax.axis_index("s")
    b = 4 * c + s // 4
    hg = s % 4
    r0 = b * 12 + 3 * hg

    io = _iota()
    pltpu.sync_copy(x_hbm.at[pl.ds(r0 * N, 3 * N)], rows_v)

    zero16 = jnp.zeros((L,), jnp.float32)
    for j in range(NCHUNK):
        cnt_v[pl.ds(j * L, L)] = zero16

    for rr in range(3):
        chunks = [rows_v[pl.ds(rr * N + j * L, L)] for j in range(NCHUNK)]
        r1, r2 = _top32_tree(chunks)
        thr = r2[K - L - 1]
        tsp = jnp.full((L,), thr, jnp.float32)

        gcnt = (jnp.where(r1 > tsp, 1, 0) + jnp.where(r2 > tsp, 1, 0))
        need = K - _splat_sum_i(gcnt, redi_v)[0]

        def fcond(st):
            i, _ = st
            return i < need

        def fbody(st):
            i, cc = st
            csp = jnp.full((L,), cc, jnp.int32)
            bidx = jnp.full((L,), BIG, jnp.int32)
            for j in range(NCHUNK):
                v = rows_v[pl.ds(rr * N + j * L, L)]
                p = io + j * L
                hit = jnp.logical_and(v == tsp, p > csp)
                bidx = jnp.minimum(bidx, jnp.where(hit, p, BIG))
            return i + 1, _splat_min_i(bidx, redi_v)[0]

        _, cut = lax.while_loop(fcond, fbody, (jnp.int32(0), jnp.int32(-1)))
        csp = jnp.full((L,), cut, jnp.int32)

        for j in range(NCHUNK):
            v = rows_v[pl.ds(rr * N + j * L, L)]
            p = io + j * L
            memb = jnp.logical_or(
                v > tsp, jnp.logical_and(v == tsp, p <= csp))
            cnt_v[pl.ds(j * L, L)] = (cnt_v[pl.ds(j * L, L)]
                                      + jnp.where(memb, 1.0, 0.0))

    pltpu.sync_copy(cnt_v, shared.at[pl.ds(s * N, N)])
    plsc.subcore_barrier()

    @pl.when(s % 4 == 0)
    def _leader():
        pltpu.sync_copy(shared.at[pl.ds(s * N, 4 * N)], four_v)
        for j in range(NCHUNK):
            base = j * L
            t0 = four_v[pl.ds(base, L)] + four_v[pl.ds(N + base, L)]
            t1 = (four_v[pl.ds(2 * N + base, L)]
                  + four_v[pl.ds(3 * N + base, L)])
            tot_v[pl.ds(base, L)] = t0 + t1

        for j in range(NCHUNK):
            base = j * L
            p = io + base
            xc = p % 24
            t = tot_v[pl.ds(base, L)]
            left = plsc.load_gather(tot_v, [jnp.maximum(p - 1, 0)])
            right = plsc.load_gather(tot_v, [jnp.minimum(p + 1, N - 1)])
            h_v[pl.ds(base, L)] = (2.0 * t
                                   + jnp.where(xc > 0, left, 0.0)
                                   + jnp.where(xc < 23, right, 0.0))

        for j in range(NCHUNK):
            base = j * L
            p = io + base
            t = h_v[pl.ds(base, L)]
            up = plsc.load_gather(h_v, [jnp.maximum(p - 24, 0)])
            dn = plsc.load_gather(h_v, [jnp.minimum(p + 24, N - 1)])
            e_v[pl.ds(base, L)] = (2.0 * t
                                   + jnp.where(p >= 24, up, 0.0)
                                   + jnp.where(p < N - 24, dn, 0.0))

        keys = []
        for j in range(NCHUNK):
            base = j * L
            p = io + base
            ei = e_v[pl.ds(base, L)].astype(jnp.int32)
            keys.append(jnp.bitwise_or(jnp.left_shift(ei, 10), 1023 - p))
        k1, k2 = _top32_tree(keys)
        o0 = 1024 - jnp.bitwise_and(k1, 1023)
        o1 = 1024 - jnp.bitwise_and(k2, 1023)
        out_v[pl.ds(0, L)] = o0
        out_v[pl.ds(L, L)] = o1
        pltpu.sync_copy(out_v, out_hbm.at[pl.ds(b * 32, 32)])


@jax.jit
def kernel(x):
    bb, hh, mm, _ = x.shape
    mesh = plsc.VectorSubcoreMesh(core_axis_name="c", subcore_axis_name="s")
    run = pl.kernel(
        _sc_body,
        mesh=mesh,
        compiler_params=pltpu.CompilerParams(needs_layout_passes=False),
        out_type=jax.ShapeDtypeStruct((bb * 32,), jnp.int32),
        scratch_types=[
            pltpu.VMEM((8,), jnp.float32),
            pltpu.VMEM((3 * N,), jnp.float32),
            pltpu.VMEM((N,), jnp.float32),
            pltpu.VMEM((N,), jnp.float32),
            pltpu.VMEM((N,), jnp.float32),
            pltpu.VMEM((N,), jnp.float32),
            pltpu.VMEM((32,), jnp.int32),
            pltpu.VMEM((4 * N,), jnp.float32),
            pltpu.VMEM((L,), jnp.int32),
            pltpu.VMEM_SHARED((16 * N,), jnp.float32),
        ],
    )
    score = x[:, :, 0, 1:].reshape(bb * hh * (mm - 1))
    return run(score).reshape(bb, 32)[:, :K]
